# baseline (speedup 1.0000x reference)
import jax
import jax.numpy as jnp
from jax import lax
from jax.experimental import pallas as pl
from jax.experimental.pallas import tpu as pltpu

_F, _V, _D_NAME, _D_VAL = 100, 100, 16, 16
_OUT_D = _D_NAME + _D_VAL
_ROW = _F * _OUT_D
_BBLK = 512


def _emb_kernel(fv_ref, tab_ref, w_ref, b_ref, idx_ref, val_ref, np_ref):
    idxs = idx_ref[...]
    vio = lax.broadcasted_iota(jnp.int32, (_V, _F), 0)
    oh_t = (vio == idxs).astype(jnp.float32)
    name_emb = lax.dot_general(
        oh_t, tab_ref[...], (((0,), (0,)), ((), ())),
        preferred_element_type=jnp.float32)
    bias = jnp.broadcast_to(b_ref[...], (_F, _D_VAL))
    np_ref[...] = jnp.concatenate([name_emb, bias], axis=1)

    jio = lax.broadcasted_iota(jnp.int32, (_F, _ROW), 1)
    fio = lax.broadcasted_iota(jnp.int32, (_F, _ROW), 0)
    e_mat = ((jio // _OUT_D) == fio).astype(jnp.float32)
    jio2 = lax.broadcasted_iota(jnp.int32, (_OUT_D, _ROW), 1)
    cio = lax.broadcasted_iota(jnp.int32, (_OUT_D, _ROW), 0)
    g_mat = ((jio2 % _OUT_D) == cio).astype(jnp.float32)
    scale = jnp.concatenate(
        [jnp.zeros((1, _D_NAME), jnp.float32), w_ref[...].T], axis=1)
    scale_row = lax.dot_general(
        scale, g_mat, (((1,), (0,)), ((), ())),
        preferred_element_type=jnp.float32)
    a_mat = e_mat * scale_row

    val_ref[...] = lax.dot_general(
        fv_ref[...], a_mat, (((1,), (0,)), ((), ())),
        preferred_element_type=jnp.float32)


def kernel(feature_values, name_table, W, b, name_indices):
    batch = feature_values.shape[0]
    b2 = b.reshape(1, _D_VAL)
    idx2 = name_indices.reshape(1, _F).astype(jnp.int32)
    val2d, name_part = pl.pallas_call(
        _emb_kernel,
        grid=(batch // _BBLK,),
        in_specs=[
            pl.BlockSpec((_BBLK, _F), lambda i: (i, 0)),
            pl.BlockSpec((_V, _D_NAME), lambda i: (0, 0)),
            pl.BlockSpec((_D_VAL, 1), lambda i: (0, 0)),
            pl.BlockSpec((1, _D_VAL), lambda i: (0, 0)),
            pl.BlockSpec((1, _F), lambda i: (0, 0)),
        ],
        out_specs=[
            pl.BlockSpec((_BBLK, _ROW), lambda i: (i, 0)),
            pl.BlockSpec((_F, _OUT_D), lambda i: (0, 0)),
        ],
        out_shape=[
            jax.ShapeDtypeStruct((batch, _ROW), jnp.float32),
            jax.ShapeDtypeStruct((_F, _OUT_D), jnp.float32),
        ],
    )(feature_values, name_table, W, b2, idx2)
    return val2d.reshape(batch, _F, _OUT_D) + name_part[None, :, :]


# speedup vs baseline: 1.4661x; 1.4661x over previous
import jax
import jax.numpy as jnp
from jax import lax
from jax.experimental import pallas as pl

_F, _V, _D_NAME, _D_VAL = 100, 100, 16, 16
_OUT_D = _D_NAME + _D_VAL
_ROW = _F * _OUT_D
_BBLK = 512


def _emb_kernel(fv_ref, tab_ref, w_ref, b_ref, idx_ref, out_ref):
    idxs = idx_ref[...]
    vio = lax.broadcasted_iota(jnp.int32, (_V, _F), 0)
    oh_t = (vio == idxs).astype(jnp.float32)
    name_emb = lax.dot_general(
        oh_t, tab_ref[...], (((0,), (0,)), ((), ())),
        preferred_element_type=jnp.float32)
    bias = jnp.broadcast_to(b_ref[...], (_F, _D_VAL))
    name_part = jnp.concatenate([name_emb, bias], axis=1)

    jio = lax.broadcasted_iota(jnp.int32, (_F, _ROW), 1)
    fio = lax.broadcasted_iota(jnp.int32, (_F, _ROW), 0)
    e_mat = ((jio // _OUT_D) == fio).astype(jnp.float32)
    jio2 = lax.broadcasted_iota(jnp.int32, (_OUT_D, _ROW), 1)
    cio = lax.broadcasted_iota(jnp.int32, (_OUT_D, _ROW), 0)
    g_mat = ((jio2 % _OUT_D) == cio).astype(jnp.float32)

    np_exp = lax.dot_general(
        name_part, e_mat, (((0,), (0,)), ((), ())),
        preferred_element_type=jnp.float32)
    name_row = jnp.sum(g_mat * np_exp, axis=0, keepdims=True)

    scale = jnp.concatenate(
        [jnp.zeros((1, _D_NAME), jnp.float32), w_ref[...].T], axis=1)
    scale_row = lax.dot_general(
        scale, g_mat, (((1,), (0,)), ((), ())),
        preferred_element_type=jnp.float32)
    a_mat = e_mat * scale_row

    out_ref[...] = lax.dot_general(
        fv_ref[...], a_mat, (((1,), (0,)), ((), ())),
        preferred_element_type=jnp.float32) + name_row


def kernel(feature_values, name_table, W, b, name_indices):
    batch = feature_values.shape[0]
    b2 = b.reshape(1, _D_VAL)
    idx2 = name_indices.reshape(1, _F).astype(jnp.int32)
    out = pl.pallas_call(
        _emb_kernel,
        grid=(batch // _BBLK,),
        in_specs=[
            pl.BlockSpec((_BBLK, _F), lambda i: (i, 0)),
            pl.BlockSpec((_V, _D_NAME), lambda i: (0, 0)),
            pl.BlockSpec((_D_VAL, 1), lambda i: (0, 0)),
            pl.BlockSpec((1, _D_VAL), lambda i: (0, 0)),
            pl.BlockSpec((1, _F), lambda i: (0, 0)),
        ],
        out_specs=pl.BlockSpec((_BBLK, _ROW), lambda i: (i, 0)),
        out_shape=jax.ShapeDtypeStruct((batch, _ROW), jnp.float32),
    )(feature_values, name_table, W, b2, idx2)
    return out.reshape(batch, _F, _OUT_D)
